# baseline (speedup 1.0000x reference)
import functools

import jax
import jax.numpy as jnp
from jax import lax
from jax.experimental import pallas as pl
from jax.experimental.pallas import tpu as pltpu
from jax.experimental.pallas import tpu_sc as plsc

B = 16384
D = 64
NC = 2
NS = 16
NW = NC * NS
BPW = B // NW
ICH = 128
NCH = BPW // ICH
BLK = 16
NBLK = BPW // BLK
CCH = 32


def _sc_body(ft_hbm, w_hbm, t_hbm, b_hbm, out_hbm,
             idx_v, w_v, w2_v, f_v, bias_v, part_v, sem, semw, semw2):
    c = lax.axis_index("c")
    s = lax.axis_index("s")
    wid = s * NC + c
    base = wid * BPW

    pltpu.sync_copy(t_hbm.at[pl.ds(base, BPW)], idx_v)

    copies = []
    for k in range(NCH):
        sl = pl.ds(k * ICH, ICH)
        copies.append(pltpu.async_copy(b_hbm.at[idx_v.at[sl]],
                                       bias_v.at[sl], sem))
    copies.append(pltpu.async_copy(ft_hbm.at[:, pl.ds(base, BPW)], f_v, sem))

    lanes = lax.iota(jnp.int32, BLK)
    NCHK = BPW // CCH

    def fire(ch, buf, semx):
        for g in range(CCH // BLK):
            tv = idx_v[pl.ds(ch * CCH + g * BLK, BLK)]
            for l in range(BLK):
                blk = lax.shift_right_logical(tv[l], 3)
                pltpu.async_copy(w_hbm.at[pl.ds(blk, 1)],
                                 buf.at[pl.ds(g * BLK + l, 1)], semx)

    def drain_compute(ch, buf, semx):
        for _ in range(CCH):
            pltpu.make_async_copy(w_hbm.at[pl.ds(0, 1)],
                                  buf.at[pl.ds(0, 1)], semx).wait()
        for g in range(CCH // BLK):
            col = ch * CCH + g * BLK
            slot = g * BLK + lanes
            rsub = idx_v[pl.ds(col, BLK)] & 7
            dots = [jnp.zeros((BLK,), jnp.float32) for _ in range(4)]
            for j in range(D):
                wcol = plsc.load_gather(
                    buf, [slot, rsub, jnp.full((BLK,), j, jnp.int32)])
                dots[j % 4] = dots[j % 4] + wcol * f_v[j, pl.ds(col, BLK)]
            dot = (dots[0] + dots[1]) + (dots[2] + dots[3])
            d = dot + bias_v[pl.ds(col, BLK)] - 1.0
            part_v[...] = part_v[...] + d * d

    part_v[...] = jnp.zeros((BLK,), jnp.float32)
    fire(0, w_v, semw)
    fire(1, w2_v, semw2)
    for cp in copies:
        cp.wait()

    def chunk_body(ch, carry):
        nx = ch + 2
        even = (ch & 1) == 0

        @pl.when(even)
        def _():
            drain_compute(ch, w_v, semw)

        @pl.when(jnp.logical_and(even, nx < NCHK))
        def _():
            fire(nx, w_v, semw)

        @pl.when(jnp.logical_not(even))
        def _():
            drain_compute(ch, w2_v, semw2)

        @pl.when(jnp.logical_and(jnp.logical_not(even), nx < NCHK))
        def _():
            fire(nx, w2_v, semw2)

        return carry

    lax.fori_loop(0, NCHK, chunk_body, 0)
    pltpu.sync_copy(part_v, out_hbm.at[pl.ds(wid * BLK, BLK)])


_sc_call = functools.partial(
    pl.kernel,
    out_type=jax.ShapeDtypeStruct((NW * BLK,), jnp.float32),
    mesh=plsc.VectorSubcoreMesh(core_axis_name="c", subcore_axis_name="s"),
    compiler_params=pltpu.CompilerParams(
        needs_layout_passes=False, use_tc_tiling_on_sc=True
    ),
    scratch_types=[
        pltpu.VMEM((BPW,), jnp.int32),
        pltpu.VMEM((CCH, 8, D), jnp.float32),
        pltpu.VMEM((CCH, 8, D), jnp.float32),
        pltpu.VMEM((D, BPW), jnp.float32),
        pltpu.VMEM((BPW,), jnp.float32),
        pltpu.VMEM((BLK,), jnp.float32),
        pltpu.SemaphoreType.DMA,
        pltpu.SemaphoreType.DMA,
        pltpu.SemaphoreType.DMA,
    ],
)(_sc_body)


def kernel(features, W, targets, b):
    t32 = targets.astype(jnp.int32)
    parts = _sc_call(features.T, W.reshape(-1, 8, D), t32, b)
    return jnp.sum(parts) * (0.5 / B)
